# Initial kernel scaffold; baseline (speedup 1.0000x reference)
import jax
import jax.numpy as jnp
from jax.experimental import pallas as pl


def kernel(h, edge_index, coord, We1, be1, We2, be2, Wn1, bn1, Wn2, bn2, Wc1, bc1, Wc2, Wv1, bv1, Wv2, bv2):
    raise NotImplementedError("write your pallas kernel here")


# speedup vs baseline: 6.7387x; 6.7387x over previous
import functools

import jax
import jax.numpy as jnp
from jax import lax
from jax.experimental import pallas as pl
from jax.experimental.pallas import tpu as pltpu
from jax.experimental.pallas import tpu_sc as plsc

N_NODES = 10000
E_EDGES = 320000
D_DIM = 128
H_DIM = 64

NC, NS = 2, 16
NW = NC * NS
EPW = E_EDGES // NW
CG = 400
CS = 200
CP = 16
PW = 128
NP_NODES = 10240
RPT = NP_NODES // NS
NB = 2000
EB = 2000

_f32 = jnp.float32


def _pre_body(h_ref, cp_ref, we1a, we1b, wn1a, wv1, bv1, wv2, bv2,
              tr_ref, tb_ref, hn_ref, vel_ref):
    hb = h_ref[...]
    cp = cp_ref[...]
    pad = jnp.zeros((NB, PW - H_DIM - CP), _f32)
    a = jnp.dot(hb, we1a[...], preferred_element_type=_f32)
    b = jnp.dot(hb, we1b[...], preferred_element_type=_f32)
    tr_ref[...] = jnp.concatenate([a, cp, pad], axis=1)
    tb_ref[...] = jnp.concatenate([b, cp, pad], axis=1)
    hn_ref[...] = jnp.dot(hb, wn1a[...], preferred_element_type=_f32)
    t = jnp.maximum(jnp.dot(hb, wv1[...], preferred_element_type=_f32)
                    + bv1[...], 0.0)
    vel_ref[...] = jnp.dot(t, wv2[...], preferred_element_type=_f32) + bv2[...]


def _precompute(h, coord_p, we1a, we1b, wn1a, wv1, bv1, wv2, bv2):
    n = h.shape[0]
    grid = (n // NB,)
    full = lambda shape: pl.BlockSpec(shape, lambda i: (0, 0))
    return pl.pallas_call(
        _pre_body,
        grid=grid,
        in_specs=[
            pl.BlockSpec((NB, D_DIM), lambda i: (i, 0)),
            pl.BlockSpec((NB, CP), lambda i: (i, 0)),
            full((D_DIM, H_DIM)), full((D_DIM, H_DIM)), full((D_DIM, H_DIM)),
            full((D_DIM, H_DIM)), full((1, H_DIM)), full((H_DIM, 1)),
            full((1, 1)),
        ],
        out_specs=[
            pl.BlockSpec((NB, PW), lambda i: (i, 0)),
            pl.BlockSpec((NB, PW), lambda i: (i, 0)),
            pl.BlockSpec((NB, H_DIM), lambda i: (i, 0)),
            pl.BlockSpec((NB, 1), lambda i: (i, 0)),
        ],
        out_shape=[
            jax.ShapeDtypeStruct((n, PW), _f32),
            jax.ShapeDtypeStruct((n, PW), _f32),
            jax.ShapeDtypeStruct((n, H_DIM), _f32),
            jax.ShapeDtypeStruct((n, 1), _f32),
        ],
    )(h, coord_p, we1a, we1b, wn1a, wv1, bv1, wv2, bv2)


def _gather_body(tr_hbm, tb_hbm, row_hbm, col_hbm, gr_hbm, gc_hbm,
                 rowv, colv, grv, gcv, sem):
    wid = lax.axis_index("s") * NC + lax.axis_index("c")

    def chunk(k, carry):
        base = wid * EPW + k * CG
        pltpu.sync_copy(row_hbm.at[pl.ds(base, CG)], rowv)
        pltpu.sync_copy(col_hbm.at[pl.ds(base, CG)], colv)
        pltpu.async_copy(tr_hbm.at[rowv], grv, sem).wait()
        pltpu.async_copy(tb_hbm.at[colv], gcv, sem).wait()
        pltpu.sync_copy(grv, gr_hbm.at[pl.ds(base, CG)])
        pltpu.sync_copy(gcv, gc_hbm.at[pl.ds(base, CG)])
        return carry

    lax.fori_loop(0, EPW // CG, chunk, 0)


def _gather(tr_tab, tb_tab, row, col):
    gk = functools.partial(
        pl.kernel,
        out_type=(
            jax.ShapeDtypeStruct((E_EDGES, PW), _f32),
            jax.ShapeDtypeStruct((E_EDGES, PW), _f32),
        ),
        mesh=plsc.VectorSubcoreMesh(core_axis_name="c", subcore_axis_name="s",
                                    num_cores=NC, num_subcores=NS),
        scratch_types=[
            pltpu.VMEM((CG,), jnp.int32),
            pltpu.VMEM((CG,), jnp.int32),
            pltpu.VMEM((CG, PW), _f32),
            pltpu.VMEM((CG, PW), _f32),
            pltpu.SemaphoreType.DMA,
        ],
    )(_gather_body)
    return gk(tr_tab, tb_tab, row, col)


def _edge_body(gr_ref, gc_ref, we1r, be1, we2, be2, wc1, bc1, wc2, out_ref):
    gr = gr_ref[...]
    gc = gc_ref[...]
    diff = gr[:, H_DIM:H_DIM + CP] - gc[:, H_DIM:H_DIM + CP]
    radial = jnp.sum(diff * diff, axis=1, keepdims=True)
    z1 = (gr[:, 0:H_DIM] + gc[:, 0:H_DIM]
          + radial * we1r[...] + be1[...])
    x = jnp.maximum(z1, 0.0)
    ef = jnp.maximum(jnp.dot(x, we2[...], preferred_element_type=_f32)
                     + be2[...], 0.0)
    h1 = jnp.maximum(jnp.dot(ef, wc1[...], preferred_element_type=_f32)
                     + bc1[...], 0.0)
    ce = jnp.dot(h1, wc2[...], preferred_element_type=_f32)
    trans = jnp.clip(diff * ce, -100.0, 100.0)
    cnt = (lax.broadcasted_iota(jnp.int32, (EB, CP), 1) == CP - 1
           ).astype(_f32)
    pad = jnp.zeros((EB, PW - H_DIM - CP), _f32)
    out_ref[...] = jnp.concatenate([ef, trans + cnt, pad], axis=1)


def _edge_mlp(gr, gc, we1r, be1, we2, be2, wc1, bc1, wc2):
    grid = (E_EDGES // EB,)
    full = lambda shape: pl.BlockSpec(shape, lambda i: (0, 0))
    return pl.pallas_call(
        _edge_body,
        grid=grid,
        in_specs=[
            pl.BlockSpec((EB, PW), lambda i: (i, 0)),
            pl.BlockSpec((EB, PW), lambda i: (i, 0)),
            full((1, H_DIM)), full((1, H_DIM)),
            full((H_DIM, H_DIM)), full((1, H_DIM)),
            full((H_DIM, H_DIM)), full((1, H_DIM)), full((H_DIM, 1)),
        ],
        out_specs=pl.BlockSpec((EB, PW), lambda i: (i, 0)),
        out_shape=jax.ShapeDtypeStruct((E_EDGES, PW), _f32),
    )(gr, gc, we1r, be1, we2, be2, wc1, bc1, wc2)


def _scatter_body(packed_hbm, row_hbm, out_hbm, idxv, pv, shared, sem):
    cid = lax.axis_index("c")
    sid = lax.axis_index("s")
    wid = sid * NC + cid

    def zrow(i, carry):
        for j in range(PW // 16):
            pv[i, pl.ds(j * 16, 16)] = jnp.zeros((16,), _f32)
        return carry

    lax.fori_loop(0, CS, zrow, 0)

    def zcopy(i, carry):
        pltpu.sync_copy(pv.at[pl.ds(0, 160)],
                        shared.at[pl.ds(sid * RPT + i * 160, 160)])
        return carry

    lax.fori_loop(0, RPT // 160, zcopy, 0)
    plsc.subcore_barrier()

    def chunk(k, carry):
        base = wid * EPW + k * CS
        pltpu.sync_copy(row_hbm.at[pl.ds(base, CS)], idxv)
        pltpu.sync_copy(packed_hbm.at[pl.ds(base, CS)], pv)
        pltpu.async_copy(pv, shared.at[idxv], sem, add=True).wait()
        return carry

    lax.fori_loop(0, EPW // CS, chunk, 0)
    plsc.subcore_barrier()
    pltpu.sync_copy(shared.at[pl.ds(sid * RPT, RPT)],
                    out_hbm.at[cid, pl.ds(sid * RPT, RPT)])


def _scatter(packed, row):
    sk = functools.partial(
        pl.kernel,
        out_type=jax.ShapeDtypeStruct((NC, NP_NODES, PW), _f32),
        mesh=plsc.VectorSubcoreMesh(core_axis_name="c", subcore_axis_name="s",
                                    num_cores=NC, num_subcores=NS),
        scratch_types=[
            pltpu.VMEM((CS,), jnp.int32),
            pltpu.VMEM((CS, PW), _f32),
            pltpu.VMEM_SHARED((NP_NODES, PW), _f32),
            pltpu.SemaphoreType.DMA,
        ],
    )(_scatter_body)
    return sk(packed, row)


def _node_body(agg0_ref, agg1_ref, hn_ref, wn1b, bn1, wn2, bn2,
               node_ref, force_ref):
    agg = agg0_ref[...] + agg1_ref[...]
    agg_e = agg[:, 0:H_DIM]
    t = jnp.maximum(hn_ref[...]
                    + jnp.dot(agg_e, wn1b[...], preferred_element_type=_f32)
                    + bn1[...], 0.0)
    node_ref[...] = jnp.dot(t, wn2[...], preferred_element_type=_f32) + bn2[...]
    tr = agg[:, H_DIM:H_DIM + 3]
    cnt = jnp.maximum(agg[:, H_DIM + CP - 1:H_DIM + CP], 1.0)
    force_ref[...] = tr / cnt


def _node_mlp(agg0, agg1, hn, wn1b, bn1, wn2, bn2):
    grid = (N_NODES // NB,)
    full = lambda shape: pl.BlockSpec(shape, lambda i: (0, 0))
    return pl.pallas_call(
        _node_body,
        grid=grid,
        in_specs=[
            pl.BlockSpec((NB, PW), lambda i: (i, 0)),
            pl.BlockSpec((NB, PW), lambda i: (i, 0)),
            pl.BlockSpec((NB, H_DIM), lambda i: (i, 0)),
            full((H_DIM, H_DIM)), full((1, H_DIM)),
            full((H_DIM, D_DIM)), full((1, D_DIM)),
        ],
        out_specs=[
            pl.BlockSpec((NB, D_DIM), lambda i: (i, 0)),
            pl.BlockSpec((NB, 3), lambda i: (i, 0)),
        ],
        out_shape=[
            jax.ShapeDtypeStruct((N_NODES, D_DIM), _f32),
            jax.ShapeDtypeStruct((N_NODES, 3), _f32),
        ],
    )(agg0, agg1, hn, wn1b, bn1, wn2, bn2)


def kernel(h, edge_index, coord, We1, be1, We2, be2, Wn1, bn1, Wn2, bn2,
           Wc1, bc1, Wc2, Wv1, bv1, Wv2, bv2):
    row = edge_index[0].astype(jnp.int32)
    col = edge_index[1].astype(jnp.int32)

    we1a = We1[:D_DIM]
    we1b = We1[D_DIM:2 * D_DIM]
    we1r = We1[2 * D_DIM:2 * D_DIM + 1]

    coord_p = jnp.zeros((N_NODES, CP), _f32).at[:, :3].set(coord)

    tr_tab, tb_tab, hn_tab, vel = _precompute(
        h, coord_p, we1a, we1b, Wn1[:D_DIM], Wv1,
        bv1.reshape(1, H_DIM), Wv2, bv2.reshape(1, 1))

    gr, gc = _gather(tr_tab, tb_tab, row, col)

    packed = _edge_mlp(gr, gc, we1r,
                       be1.reshape(1, H_DIM), We2, be2.reshape(1, H_DIM),
                       Wc1, bc1.reshape(1, H_DIM), Wc2)

    agg = _scatter(packed, row)

    node, force = _node_mlp(agg[0], agg[1], hn_tab,
                            Wn1[D_DIM:], bn1.reshape(1, H_DIM),
                            Wn2, bn2.reshape(1, D_DIM))
    return (vel, force, node)


# speedup vs baseline: 7.9036x; 1.1729x over previous
import functools

import jax
import jax.numpy as jnp
from jax import lax
from jax.experimental import pallas as pl
from jax.experimental.pallas import tpu as pltpu
from jax.experimental.pallas import tpu_sc as plsc

N_NODES = 10000
E_EDGES = 320000
D_DIM = 128
H_DIM = 64

NC, NS = 2, 16
NW = NC * NS
EPW = E_EDGES // NW
CG = 200
CS = 200
CP = 16
PW = 128
NP_NODES = 10240
RPT = NP_NODES // NS
NB = 2000
EB = 2000

_f32 = jnp.float32


def _pre_body(h_ref, cp_ref, we1a, we1b, wn1a, wv1, bv1, wv2, bv2,
              tr_ref, tb_ref, hn_ref, vel_ref):
    hb = h_ref[...]
    cp = cp_ref[...]
    pad = jnp.zeros((NB, PW - H_DIM - CP), _f32)
    a = jnp.dot(hb, we1a[...], preferred_element_type=_f32)
    b = jnp.dot(hb, we1b[...], preferred_element_type=_f32)
    tr_ref[...] = jnp.concatenate([a, cp, pad], axis=1)
    tb_ref[...] = jnp.concatenate([b, -cp, pad], axis=1)
    hn_ref[...] = jnp.dot(hb, wn1a[...], preferred_element_type=_f32)
    t = jnp.maximum(jnp.dot(hb, wv1[...], preferred_element_type=_f32)
                    + bv1[...], 0.0)
    vel_ref[...] = jnp.dot(t, wv2[...], preferred_element_type=_f32) + bv2[...]


def _precompute(h, coord_p, we1a, we1b, wn1a, wv1, bv1, wv2, bv2):
    n = h.shape[0]
    grid = (n // NB,)
    full = lambda shape: pl.BlockSpec(shape, lambda i: (0, 0))
    return pl.pallas_call(
        _pre_body,
        grid=grid,
        in_specs=[
            pl.BlockSpec((NB, D_DIM), lambda i: (i, 0)),
            pl.BlockSpec((NB, CP), lambda i: (i, 0)),
            full((D_DIM, H_DIM)), full((D_DIM, H_DIM)), full((D_DIM, H_DIM)),
            full((D_DIM, H_DIM)), full((1, H_DIM)), full((H_DIM, 1)),
            full((1, 1)),
        ],
        out_specs=[
            pl.BlockSpec((NB, PW), lambda i: (i, 0)),
            pl.BlockSpec((NB, PW), lambda i: (i, 0)),
            pl.BlockSpec((NB, H_DIM), lambda i: (i, 0)),
            pl.BlockSpec((NB, 1), lambda i: (i, 0)),
        ],
        out_shape=[
            jax.ShapeDtypeStruct((n, PW), _f32),
            jax.ShapeDtypeStruct((n, PW), _f32),
            jax.ShapeDtypeStruct((n, H_DIM), _f32),
            jax.ShapeDtypeStruct((n, 1), _f32),
        ],
    )(h, coord_p, we1a, we1b, wn1a, wv1, bv1, wv2, bv2)


def _gather_body(tr_hbm, tb_hbm, row_hbm, col_hbm, g_hbm,
                 rowv0, colv0, grv0, gcv0, rowv1, colv1, grv1, gcv1,
                 semg0, semg1, semw0, semw1):
    wid = lax.axis_index("s") * NC + lax.axis_index("c")
    nch = EPW // CG
    slots = ((rowv0, colv0, grv0, gcv0, semg0, semw0),
             (rowv1, colv1, grv1, gcv1, semg1, semw1))

    def issue(k, slot, wait_wb):
        rowv, colv, grv, gcv, semg, semw = slot
        base = wid * EPW + k * CG
        pltpu.sync_copy(row_hbm.at[pl.ds(base, CG)], rowv)
        pltpu.sync_copy(col_hbm.at[pl.ds(base, CG)], colv)
        if wait_wb:
            pltpu.make_async_copy(gcv, g_hbm.at[pl.ds(base, CG)], semw).wait()
        pltpu.async_copy(tr_hbm.at[rowv], grv, semg)
        pltpu.async_copy(tb_hbm.at[colv], gcv, semg)

    def finish(k, slot):
        rowv, colv, grv, gcv, semg, semw = slot
        base = wid * EPW + k * CG
        pltpu.make_async_copy(tr_hbm.at[rowv], grv, semg).wait()
        pltpu.make_async_copy(tb_hbm.at[colv], gcv, semg).wait()

        def add_row(r, carry):
            for j in range(PW // 16):
                sl = pl.ds(j * 16, 16)
                gcv[r, sl] = grv[r, sl] + gcv[r, sl]
            return carry

        lax.fori_loop(0, CG, add_row, 0)
        pltpu.async_copy(gcv, g_hbm.at[pl.ds(base, CG)], semw)

    issue(0, slots[0], False)
    issue(1, slots[1], False)

    def pair(g, carry):
        finish(2 * g, slots[0])
        issue(2 * g + 2, slots[0], True)
        finish(2 * g + 1, slots[1])
        issue(2 * g + 3, slots[1], True)
        return carry

    lax.fori_loop(0, nch // 2 - 1, pair, 0)
    finish(nch - 2, slots[0])
    finish(nch - 1, slots[1])
    pltpu.make_async_copy(
        gcv0, g_hbm.at[pl.ds(0, CG)], semw0).wait()
    pltpu.make_async_copy(
        gcv1, g_hbm.at[pl.ds(0, CG)], semw1).wait()


def _gather(tr_tab, tb_tab, row, col):
    gk = functools.partial(
        pl.kernel,
        out_type=jax.ShapeDtypeStruct((E_EDGES, PW), _f32),
        mesh=plsc.VectorSubcoreMesh(core_axis_name="c", subcore_axis_name="s",
                                    num_cores=NC, num_subcores=NS),
        scratch_types=[
            pltpu.VMEM((CG,), jnp.int32),
            pltpu.VMEM((CG,), jnp.int32),
            pltpu.VMEM((CG, PW), _f32),
            pltpu.VMEM((CG, PW), _f32),
            pltpu.VMEM((CG,), jnp.int32),
            pltpu.VMEM((CG,), jnp.int32),
            pltpu.VMEM((CG, PW), _f32),
            pltpu.VMEM((CG, PW), _f32),
            pltpu.SemaphoreType.DMA,
            pltpu.SemaphoreType.DMA,
            pltpu.SemaphoreType.DMA,
            pltpu.SemaphoreType.DMA,
        ],
    )(_gather_body)
    return gk(tr_tab, tb_tab, row, col)


def _edge_body(g_ref, we1r, be1, we2, be2, wc1, bc1, wc2, out_ref):
    g = g_ref[...]
    diff = g[:, H_DIM:H_DIM + CP]
    radial = jnp.sum(diff * diff, axis=1, keepdims=True)
    z1 = g[:, 0:H_DIM] + radial * we1r[...] + be1[...]
    x = jnp.maximum(z1, 0.0)
    ef = jnp.maximum(jnp.dot(x, we2[...], preferred_element_type=_f32)
                     + be2[...], 0.0)
    h1 = jnp.maximum(jnp.dot(ef, wc1[...], preferred_element_type=_f32)
                     + bc1[...], 0.0)
    ce = jnp.dot(h1, wc2[...], preferred_element_type=_f32)
    trans = jnp.clip(diff * ce, -100.0, 100.0)
    cnt = (lax.broadcasted_iota(jnp.int32, (EB, CP), 1) == CP - 1
           ).astype(_f32)
    pad = jnp.zeros((EB, PW - H_DIM - CP), _f32)
    out_ref[...] = jnp.concatenate([ef, trans + cnt, pad], axis=1)


def _edge_mlp(g, we1r, be1, we2, be2, wc1, bc1, wc2):
    grid = (E_EDGES // EB,)
    full = lambda shape: pl.BlockSpec(shape, lambda i: (0, 0))
    return pl.pallas_call(
        _edge_body,
        grid=grid,
        in_specs=[
            pl.BlockSpec((EB, PW), lambda i: (i, 0)),
            full((1, H_DIM)), full((1, H_DIM)),
            full((H_DIM, H_DIM)), full((1, H_DIM)),
            full((H_DIM, H_DIM)), full((1, H_DIM)), full((H_DIM, 1)),
        ],
        out_specs=pl.BlockSpec((EB, PW), lambda i: (i, 0)),
        out_shape=jax.ShapeDtypeStruct((E_EDGES, PW), _f32),
    )(g, we1r, be1, we2, be2, wc1, bc1, wc2)


def _scatter_body(packed_hbm, row_hbm, out_hbm, idxv, pv, shared, sem):
    cid = lax.axis_index("c")
    sid = lax.axis_index("s")
    wid = sid * NC + cid

    def zrow(i, carry):
        for j in range(PW // 16):
            pv[i, pl.ds(j * 16, 16)] = jnp.zeros((16,), _f32)
        return carry

    lax.fori_loop(0, CS, zrow, 0)

    def zcopy(i, carry):
        pltpu.sync_copy(pv.at[pl.ds(0, 160)],
                        shared.at[pl.ds(sid * RPT + i * 160, 160)])
        return carry

    lax.fori_loop(0, RPT // 160, zcopy, 0)
    plsc.subcore_barrier()

    def chunk(k, carry):
        base = wid * EPW + k * CS
        pltpu.sync_copy(row_hbm.at[pl.ds(base, CS)], idxv)
        pltpu.sync_copy(packed_hbm.at[pl.ds(base, CS)], pv)
        pltpu.async_copy(pv, shared.at[idxv], sem, add=True).wait()
        return carry

    lax.fori_loop(0, EPW // CS, chunk, 0)
    plsc.subcore_barrier()
    pltpu.sync_copy(shared.at[pl.ds(sid * RPT, RPT)],
                    out_hbm.at[cid, pl.ds(sid * RPT, RPT)])


def _scatter(packed, row):
    sk = functools.partial(
        pl.kernel,
        out_type=jax.ShapeDtypeStruct((NC, NP_NODES, PW), _f32),
        mesh=plsc.VectorSubcoreMesh(core_axis_name="c", subcore_axis_name="s",
                                    num_cores=NC, num_subcores=NS),
        scratch_types=[
            pltpu.VMEM((CS,), jnp.int32),
            pltpu.VMEM((CS, PW), _f32),
            pltpu.VMEM_SHARED((NP_NODES, PW), _f32),
            pltpu.SemaphoreType.DMA,
        ],
    )(_scatter_body)
    return sk(packed, row)


def _node_body(agg0_ref, agg1_ref, hn_ref, wn1b, bn1, wn2, bn2,
               node_ref, force_ref):
    agg = agg0_ref[...] + agg1_ref[...]
    agg_e = agg[:, 0:H_DIM]
    t = jnp.maximum(hn_ref[...]
                    + jnp.dot(agg_e, wn1b[...], preferred_element_type=_f32)
                    + bn1[...], 0.0)
    node_ref[...] = jnp.dot(t, wn2[...], preferred_element_type=_f32) + bn2[...]
    tr = agg[:, H_DIM:H_DIM + 3]
    cnt = jnp.maximum(agg[:, H_DIM + CP - 1:H_DIM + CP], 1.0)
    force_ref[...] = tr / cnt


def _node_mlp(agg0, agg1, hn, wn1b, bn1, wn2, bn2):
    grid = (N_NODES // NB,)
    full = lambda shape: pl.BlockSpec(shape, lambda i: (0, 0))
    return pl.pallas_call(
        _node_body,
        grid=grid,
        in_specs=[
            pl.BlockSpec((NB, PW), lambda i: (i, 0)),
            pl.BlockSpec((NB, PW), lambda i: (i, 0)),
            pl.BlockSpec((NB, H_DIM), lambda i: (i, 0)),
            full((H_DIM, H_DIM)), full((1, H_DIM)),
            full((H_DIM, D_DIM)), full((1, D_DIM)),
        ],
        out_specs=[
            pl.BlockSpec((NB, D_DIM), lambda i: (i, 0)),
            pl.BlockSpec((NB, 3), lambda i: (i, 0)),
        ],
        out_shape=[
            jax.ShapeDtypeStruct((N_NODES, D_DIM), _f32),
            jax.ShapeDtypeStruct((N_NODES, 3), _f32),
        ],
    )(agg0, agg1, hn, wn1b, bn1, wn2, bn2)


def kernel(h, edge_index, coord, We1, be1, We2, be2, Wn1, bn1, Wn2, bn2,
           Wc1, bc1, Wc2, Wv1, bv1, Wv2, bv2):
    row = edge_index[0].astype(jnp.int32)
    col = edge_index[1].astype(jnp.int32)

    we1a = We1[:D_DIM]
    we1b = We1[D_DIM:2 * D_DIM]
    we1r = We1[2 * D_DIM:2 * D_DIM + 1]

    coord_p = jnp.zeros((N_NODES, CP), _f32).at[:, :3].set(coord)

    tr_tab, tb_tab, hn_tab, vel = _precompute(
        h, coord_p, we1a, we1b, Wn1[:D_DIM], Wv1,
        bv1.reshape(1, H_DIM), Wv2, bv2.reshape(1, 1))

    g = _gather(tr_tab, tb_tab, row, col)

    packed = _edge_mlp(g, we1r,
                       be1.reshape(1, H_DIM), We2, be2.reshape(1, H_DIM),
                       Wc1, bc1.reshape(1, H_DIM), Wc2)

    agg = _scatter(packed, row)

    node, force = _node_mlp(agg[0], agg[1], hn_tab,
                            Wn1[D_DIM:], bn1.reshape(1, H_DIM),
                            Wn2, bn2.reshape(1, D_DIM))
    return (vel, force, node)


# speedup vs baseline: 8.8103x; 1.1147x over previous
import functools

import jax
import jax.numpy as jnp
from jax import lax
from jax.experimental import pallas as pl
from jax.experimental.pallas import tpu as pltpu
from jax.experimental.pallas import tpu_sc as plsc

N_NODES = 10000
E_EDGES = 320000
D_DIM = 128
H_DIM = 64

NC, NS = 2, 16
NW = NC * NS
EPW = E_EDGES // NW
CG = 200
CS = 80
CP = 16
PW = 128
NP_NODES = 10240
RPT = NP_NODES // NS
NB = 2000
EB = 2000

_f32 = jnp.float32


def _pre_body(h_ref, cp_ref, we1a, we1b, wn1a, wv1, bv1, wv2, bv2,
              tr_ref, tb_ref, hn_ref, vel_ref):
    hb = h_ref[...]
    cp = cp_ref[...]
    pad = jnp.zeros((NB, PW - H_DIM - CP), _f32)
    a = jnp.dot(hb, we1a[...], preferred_element_type=_f32)
    b = jnp.dot(hb, we1b[...], preferred_element_type=_f32)
    tr_ref[...] = jnp.concatenate([a, cp, pad], axis=1)
    tb_ref[...] = jnp.concatenate([b, -cp, pad], axis=1)
    hn_ref[...] = jnp.dot(hb, wn1a[...], preferred_element_type=_f32)
    t = jnp.maximum(jnp.dot(hb, wv1[...], preferred_element_type=_f32)
                    + bv1[...], 0.0)
    vel_ref[...] = jnp.dot(t, wv2[...], preferred_element_type=_f32) + bv2[...]


def _precompute(h, coord_p, we1a, we1b, wn1a, wv1, bv1, wv2, bv2):
    n = h.shape[0]
    grid = (n // NB,)
    full = lambda shape: pl.BlockSpec(shape, lambda i: (0, 0))
    return pl.pallas_call(
        _pre_body,
        grid=grid,
        in_specs=[
            pl.BlockSpec((NB, D_DIM), lambda i: (i, 0)),
            pl.BlockSpec((NB, CP), lambda i: (i, 0)),
            full((D_DIM, H_DIM)), full((D_DIM, H_DIM)), full((D_DIM, H_DIM)),
            full((D_DIM, H_DIM)), full((1, H_DIM)), full((H_DIM, 1)),
            full((1, 1)),
        ],
        out_specs=[
            pl.BlockSpec((NB, PW), lambda i: (i, 0)),
            pl.BlockSpec((NB, PW), lambda i: (i, 0)),
            pl.BlockSpec((NB, H_DIM), lambda i: (i, 0)),
            pl.BlockSpec((NB, 1), lambda i: (i, 0)),
        ],
        out_shape=[
            jax.ShapeDtypeStruct((n, PW), _f32),
            jax.ShapeDtypeStruct((n, PW), _f32),
            jax.ShapeDtypeStruct((n, H_DIM), _f32),
            jax.ShapeDtypeStruct((n, 1), _f32),
        ],
    )(h, coord_p, we1a, we1b, wn1a, wv1, bv1, wv2, bv2)


def _gather_body(tr_hbm, tb_hbm, row_hbm, col_hbm, g_hbm,
                 rowv0, colv0, grv0, gcv0, rowv1, colv1, grv1, gcv1,
                 semg0, semg1, semw0, semw1):
    wid = lax.axis_index("s") * NC + lax.axis_index("c")
    nch = EPW // CG
    slots = ((rowv0, colv0, grv0, gcv0, semg0, semw0),
             (rowv1, colv1, grv1, gcv1, semg1, semw1))

    def issue(k, slot, wait_wb):
        rowv, colv, grv, gcv, semg, semw = slot
        base = wid * EPW + k * CG
        pltpu.sync_copy(row_hbm.at[pl.ds(base, CG)], rowv)
        pltpu.sync_copy(col_hbm.at[pl.ds(base, CG)], colv)
        if wait_wb:
            pltpu.make_async_copy(gcv, g_hbm.at[pl.ds(base, CG)], semw).wait()
        pltpu.async_copy(tr_hbm.at[rowv], grv, semg)
        pltpu.async_copy(tb_hbm.at[colv], gcv, semg)

    def finish(k, slot):
        rowv, colv, grv, gcv, semg, semw = slot
        base = wid * EPW + k * CG
        pltpu.make_async_copy(tr_hbm.at[rowv], grv, semg).wait()
        pltpu.make_async_copy(tb_hbm.at[colv], gcv, semg).wait()

        def add_row(r, carry):
            for j in range(PW // 16):
                sl = pl.ds(j * 16, 16)
                gcv[r, sl] = grv[r, sl] + gcv[r, sl]
            return carry

        lax.fori_loop(0, CG, add_row, 0)
        pltpu.async_copy(gcv, g_hbm.at[pl.ds(base, CG)], semw)

    issue(0, slots[0], False)
    issue(1, slots[1], False)

    def pair(g, carry):
        finish(2 * g, slots[0])
        issue(2 * g + 2, slots[0], True)
        finish(2 * g + 1, slots[1])
        issue(2 * g + 3, slots[1], True)
        return carry

    lax.fori_loop(0, nch // 2 - 1, pair, 0)
    finish(nch - 2, slots[0])
    finish(nch - 1, slots[1])
    pltpu.make_async_copy(
        gcv0, g_hbm.at[pl.ds(0, CG)], semw0).wait()
    pltpu.make_async_copy(
        gcv1, g_hbm.at[pl.ds(0, CG)], semw1).wait()


def _gather(tr_tab, tb_tab, row, col):
    gk = functools.partial(
        pl.kernel,
        out_type=jax.ShapeDtypeStruct((E_EDGES, PW), _f32),
        mesh=plsc.VectorSubcoreMesh(core_axis_name="c", subcore_axis_name="s",
                                    num_cores=NC, num_subcores=NS),
        scratch_types=[
            pltpu.VMEM((CG,), jnp.int32),
            pltpu.VMEM((CG,), jnp.int32),
            pltpu.VMEM((CG, PW), _f32),
            pltpu.VMEM((CG, PW), _f32),
            pltpu.VMEM((CG,), jnp.int32),
            pltpu.VMEM((CG,), jnp.int32),
            pltpu.VMEM((CG, PW), _f32),
            pltpu.VMEM((CG, PW), _f32),
            pltpu.SemaphoreType.DMA,
            pltpu.SemaphoreType.DMA,
            pltpu.SemaphoreType.DMA,
            pltpu.SemaphoreType.DMA,
        ],
    )(_gather_body)
    return gk(tr_tab, tb_tab, row, col)


def _edge_body(g_ref, we1rb, be1, we2, be2, wc1, bc1, wc2, cntrow, out_ref):
    g = g_ref[...]
    diff = g[:, H_DIM:H_DIM + CP]
    d2 = diff * diff
    z1 = (g[:, 0:H_DIM]
          + jnp.dot(d2, we1rb[...], preferred_element_type=_f32) + be1[...])
    x = jnp.maximum(z1, 0.0)
    ef = jnp.maximum(jnp.dot(x, we2[...], preferred_element_type=_f32)
                     + be2[...], 0.0)
    h1 = jnp.maximum(jnp.dot(ef, wc1[...], preferred_element_type=_f32)
                     + bc1[...], 0.0)
    ce = jnp.dot(h1, wc2[...], preferred_element_type=_f32)
    out_ref[:, 0:H_DIM] = ef
    out_ref[:, H_DIM:H_DIM + CP] = (jnp.clip(diff * ce, -100.0, 100.0)
                                    + cntrow[...])


def _edge_mlp(g, we1rb, be1, we2, be2, wc1, bc1, wc2, cntrow):
    grid = (E_EDGES // EB,)
    full = lambda shape: pl.BlockSpec(shape, lambda i: (0, 0))
    return pl.pallas_call(
        _edge_body,
        grid=grid,
        in_specs=[
            pl.BlockSpec((EB, PW), lambda i: (i, 0)),
            full((CP, H_DIM)), full((1, H_DIM)),
            full((H_DIM, H_DIM)), full((1, H_DIM)),
            full((H_DIM, H_DIM)), full((1, H_DIM)), full((H_DIM, 1)),
            full((1, CP)),
        ],
        out_specs=pl.BlockSpec((EB, PW), lambda i: (i, 0)),
        out_shape=jax.ShapeDtypeStruct((E_EDGES, PW), _f32),
    )(g, we1rb, be1, we2, be2, wc1, bc1, wc2, cntrow)


def _scatter_body(packed_hbm, row_hbm, out_hbm,
                  idxv0, pv0, idxv1, pv1, shared,
                  seml0, seml1, sema0, sema1):
    cid = lax.axis_index("c")
    sid = lax.axis_index("s")
    wid = sid * NC + cid
    nch = EPW // CS
    slots = ((idxv0, pv0, seml0, sema0), (idxv1, pv1, seml1, sema1))

    def zrow(i, carry):
        for j in range(PW // 16):
            pv0[i, pl.ds(j * 16, 16)] = jnp.zeros((16,), _f32)
        return carry

    lax.fori_loop(0, CS, zrow, 0)

    def zcopy(i, carry):
        pltpu.sync_copy(pv0.at[pl.ds(0, CS)],
                        shared.at[pl.ds(sid * RPT + i * CS, CS)])
        return carry

    lax.fori_loop(0, RPT // CS, zcopy, 0)
    plsc.subcore_barrier()

    def issue(k, slot, wait_add):
        idxv, pv, seml, sema = slot
        base = wid * EPW + k * CS
        if wait_add:
            pltpu.make_async_copy(pv, shared.at[idxv], sema).wait()
        pltpu.sync_copy(row_hbm.at[pl.ds(base, CS)], idxv)
        pltpu.async_copy(packed_hbm.at[pl.ds(base, CS)], pv, seml)

    def proc(k, slot):
        idxv, pv, seml, sema = slot
        base = wid * EPW + k * CS
        pltpu.make_async_copy(packed_hbm.at[pl.ds(base, CS)], pv, seml).wait()
        pltpu.async_copy(pv, shared.at[idxv], sema, add=True)

    issue(0, slots[0], False)
    issue(1, slots[1], False)

    npair = (nch - 1) // 2 - 1

    def pair(g, carry):
        proc(2 * g, slots[0])
        issue(2 * g + 2, slots[0], True)
        proc(2 * g + 1, slots[1])
        issue(2 * g + 3, slots[1], True)
        return carry

    lax.fori_loop(0, npair, pair, 0)
    proc(2 * npair, slots[0])
    issue(2 * npair + 2, slots[0], True)
    proc(2 * npair + 1, slots[1])
    proc(2 * npair + 2, slots[0])
    pltpu.make_async_copy(pv0, shared.at[idxv0], sema0).wait()
    pltpu.make_async_copy(pv1, shared.at[idxv1], sema1).wait()
    plsc.subcore_barrier()
    pltpu.sync_copy(shared.at[pl.ds(sid * RPT, RPT)],
                    out_hbm.at[cid, pl.ds(sid * RPT, RPT)])


def _scatter(packed, row):
    sk = functools.partial(
        pl.kernel,
        out_type=jax.ShapeDtypeStruct((NC, NP_NODES, PW), _f32),
        mesh=plsc.VectorSubcoreMesh(core_axis_name="c", subcore_axis_name="s",
                                    num_cores=NC, num_subcores=NS),
        scratch_types=[
            pltpu.VMEM((CS,), jnp.int32),
            pltpu.VMEM((CS, PW), _f32),
            pltpu.VMEM((CS,), jnp.int32),
            pltpu.VMEM((CS, PW), _f32),
            pltpu.VMEM_SHARED((NP_NODES, PW), _f32),
            pltpu.SemaphoreType.DMA,
            pltpu.SemaphoreType.DMA,
            pltpu.SemaphoreType.DMA,
            pltpu.SemaphoreType.DMA,
        ],
    )(_scatter_body)
    return sk(packed, row)


def _node_body(agg0_ref, agg1_ref, hn_ref, wn1b, bn1, wn2, bn2,
               node_ref, force_ref):
    agg = agg0_ref[...] + agg1_ref[...]
    agg_e = agg[:, 0:H_DIM]
    t = jnp.maximum(hn_ref[...]
                    + jnp.dot(agg_e, wn1b[...], preferred_element_type=_f32)
                    + bn1[...], 0.0)
    node_ref[...] = jnp.dot(t, wn2[...], preferred_element_type=_f32) + bn2[...]
    tr = agg[:, H_DIM:H_DIM + 3]
    cnt = jnp.maximum(agg[:, H_DIM + CP - 1:H_DIM + CP], 1.0)
    force_ref[...] = tr / cnt


def _node_mlp(agg0, agg1, hn, wn1b, bn1, wn2, bn2):
    grid = (N_NODES // NB,)
    full = lambda shape: pl.BlockSpec(shape, lambda i: (0, 0))
    return pl.pallas_call(
        _node_body,
        grid=grid,
        in_specs=[
            pl.BlockSpec((NB, PW), lambda i: (i, 0)),
            pl.BlockSpec((NB, PW), lambda i: (i, 0)),
            pl.BlockSpec((NB, H_DIM), lambda i: (i, 0)),
            full((H_DIM, H_DIM)), full((1, H_DIM)),
            full((H_DIM, D_DIM)), full((1, D_DIM)),
        ],
        out_specs=[
            pl.BlockSpec((NB, D_DIM), lambda i: (i, 0)),
            pl.BlockSpec((NB, 3), lambda i: (i, 0)),
        ],
        out_shape=[
            jax.ShapeDtypeStruct((N_NODES, D_DIM), _f32),
            jax.ShapeDtypeStruct((N_NODES, 3), _f32),
        ],
    )(agg0, agg1, hn, wn1b, bn1, wn2, bn2)


def kernel(h, edge_index, coord, We1, be1, We2, be2, Wn1, bn1, Wn2, bn2,
           Wc1, bc1, Wc2, Wv1, bv1, Wv2, bv2):
    row = edge_index[0].astype(jnp.int32)
    col = edge_index[1].astype(jnp.int32)

    we1a = We1[:D_DIM]
    we1b = We1[D_DIM:2 * D_DIM]
    we1r = We1[2 * D_DIM:2 * D_DIM + 1]

    coord_p = jnp.zeros((N_NODES, CP), _f32).at[:, :3].set(coord)

    tr_tab, tb_tab, hn_tab, vel = _precompute(
        h, coord_p, we1a, we1b, Wn1[:D_DIM], Wv1,
        bv1.reshape(1, H_DIM), Wv2, bv2.reshape(1, 1))

    g = _gather(tr_tab, tb_tab, row, col)

    we1rb = jnp.broadcast_to(we1r, (CP, H_DIM))
    cntrow = jnp.zeros((1, CP), _f32).at[0, CP - 1].set(1.0)
    packed = _edge_mlp(g, we1rb,
                       be1.reshape(1, H_DIM), We2, be2.reshape(1, H_DIM),
                       Wc1, bc1.reshape(1, H_DIM), Wc2, cntrow)

    agg = _scatter(packed, row)

    node, force = _node_mlp(agg[0], agg[1], hn_tab,
                            Wn1[D_DIM:], bn1.reshape(1, H_DIM),
                            Wn2, bn2.reshape(1, D_DIM))
    return (vel, force, node)


# speedup vs baseline: 9.1202x; 1.0352x over previous
import functools

import jax
import jax.numpy as jnp
from jax import lax
from jax.experimental import pallas as pl
from jax.experimental.pallas import tpu as pltpu
from jax.experimental.pallas import tpu_sc as plsc

N_NODES = 10000
E_EDGES = 320000
D_DIM = 128
H_DIM = 64

NC, NS = 2, 16
NW = NC * NS
EPW = E_EDGES // NW
CG = 200
CS = 80
CP = 16
PW = 128
NP_NODES = 10240
RPT = NP_NODES // NS
NB = 2000
EB = 2000
EPW_A = 6000

_f32 = jnp.float32


def _pre_body(h_ref, cp_ref, we1a, we1b, wn1a, wv1, bv1, wv2, bv2,
              tr_ref, tb_ref, hn_ref, vel_ref):
    hb = h_ref[...]
    cp = cp_ref[...]
    pad = jnp.zeros((NB, PW - H_DIM - CP), _f32)
    a = jnp.dot(hb, we1a[...], preferred_element_type=_f32)
    b = jnp.dot(hb, we1b[...], preferred_element_type=_f32)
    tr_ref[...] = jnp.concatenate([a, cp, pad], axis=1)
    tb_ref[...] = jnp.concatenate([b, -cp, pad], axis=1)
    hn_ref[...] = jnp.dot(hb, wn1a[...], preferred_element_type=_f32)
    t = jnp.maximum(jnp.dot(hb, wv1[...], preferred_element_type=_f32)
                    + bv1[...], 0.0)
    vel_ref[...] = jnp.dot(t, wv2[...], preferred_element_type=_f32) + bv2[...]


def _precompute(h, coord_p, we1a, we1b, wn1a, wv1, bv1, wv2, bv2):
    n = h.shape[0]
    grid = (n // NB,)
    full = lambda shape: pl.BlockSpec(shape, lambda i: (0, 0))
    return pl.pallas_call(
        _pre_body,
        grid=grid,
        in_specs=[
            pl.BlockSpec((NB, D_DIM), lambda i: (i, 0)),
            pl.BlockSpec((NB, CP), lambda i: (i, 0)),
            full((D_DIM, H_DIM)), full((D_DIM, H_DIM)), full((D_DIM, H_DIM)),
            full((D_DIM, H_DIM)), full((1, H_DIM)), full((H_DIM, 1)),
            full((1, 1)),
        ],
        out_specs=[
            pl.BlockSpec((NB, PW), lambda i: (i, 0)),
            pl.BlockSpec((NB, PW), lambda i: (i, 0)),
            pl.BlockSpec((NB, H_DIM), lambda i: (i, 0)),
            pl.BlockSpec((NB, 1), lambda i: (i, 0)),
        ],
        out_shape=[
            jax.ShapeDtypeStruct((n, PW), _f32),
            jax.ShapeDtypeStruct((n, PW), _f32),
            jax.ShapeDtypeStruct((n, H_DIM), _f32),
            jax.ShapeDtypeStruct((n, 1), _f32),
        ],
    )(h, coord_p, we1a, we1b, wn1a, wv1, bv1, wv2, bv2)


def _make_gather_body(e_total):
    epw = e_total // NW
    nch = epw // CG
    npair = (nch - 2) // 2

    def body(tr_hbm, tb_hbm, row_hbm, col_hbm, g_hbm,
             rowv0, colv0, grv0, gcv0, rowv1, colv1, grv1, gcv1,
             semg0, semg1, semw0, semw1):
        wid = lax.axis_index("s") * NC + lax.axis_index("c")
        slots = ((rowv0, colv0, grv0, gcv0, semg0, semw0),
                 (rowv1, colv1, grv1, gcv1, semg1, semw1))

        def issue(k, slot, wait_wb):
            rowv, colv, grv, gcv, semg, semw = slot
            base = wid * epw + k * CG
            pltpu.sync_copy(row_hbm.at[pl.ds(base, CG)], rowv)
            pltpu.sync_copy(col_hbm.at[pl.ds(base, CG)], colv)
            if wait_wb:
                pltpu.make_async_copy(gcv, g_hbm.at[pl.ds(base, CG)],
                                      semw).wait()
            pltpu.async_copy(tr_hbm.at[rowv], grv, semg)
            pltpu.async_copy(tb_hbm.at[colv], gcv, semg)

        def finish(k, slot):
            rowv, colv, grv, gcv, semg, semw = slot
            base = wid * epw + k * CG
            pltpu.make_async_copy(tr_hbm.at[rowv], grv, semg).wait()
            pltpu.make_async_copy(tb_hbm.at[colv], gcv, semg).wait()

            def add_row(r, carry):
                for j in range(PW // 16):
                    sl = pl.ds(j * 16, 16)
                    gcv[r, sl] = grv[r, sl] + gcv[r, sl]
                return carry

            lax.fori_loop(0, CG, add_row, 0)
            pltpu.async_copy(gcv, g_hbm.at[pl.ds(base, CG)], semw)

        issue(0, slots[0], False)
        issue(1, slots[1], False)

        def pair(g, carry):
            finish(2 * g, slots[0])
            issue(2 * g + 2, slots[0], True)
            finish(2 * g + 1, slots[1])
            issue(2 * g + 3, slots[1], True)
            return carry

        lax.fori_loop(0, npair, pair, 0)
        for k in range(2 * npair, nch):
            finish(k, slots[k % 2])
            if k + 2 < nch:
                issue(k + 2, slots[k % 2], True)
        pltpu.make_async_copy(
            gcv0, g_hbm.at[pl.ds(0, CG)], semw0).wait()
        pltpu.make_async_copy(
            gcv1, g_hbm.at[pl.ds(0, CG)], semw1).wait()

    return body


def _gather(tr_tab, tb_tab, row, col):
    e_total = row.shape[0]
    gk = functools.partial(
        pl.kernel,
        out_type=jax.ShapeDtypeStruct((e_total, PW), _f32),
        mesh=plsc.VectorSubcoreMesh(core_axis_name="c", subcore_axis_name="s",
                                    num_cores=NC, num_subcores=NS),
        scratch_types=[
            pltpu.VMEM((CG,), jnp.int32),
            pltpu.VMEM((CG,), jnp.int32),
            pltpu.VMEM((CG, PW), _f32),
            pltpu.VMEM((CG, PW), _f32),
            pltpu.VMEM((CG,), jnp.int32),
            pltpu.VMEM((CG,), jnp.int32),
            pltpu.VMEM((CG, PW), _f32),
            pltpu.VMEM((CG, PW), _f32),
            pltpu.SemaphoreType.DMA,
            pltpu.SemaphoreType.DMA,
            pltpu.SemaphoreType.DMA,
            pltpu.SemaphoreType.DMA,
        ],
    )(_make_gather_body(e_total))
    return gk(tr_tab, tb_tab, row, col)


def _edge_body(g_ref, we1rb, be1, we2, be2, wc1, bc1, wc2, cntrow, out_ref):
    g = g_ref[...]
    diff = g[:, H_DIM:H_DIM + CP]
    d2 = diff * diff
    z1 = (g[:, 0:H_DIM]
          + jnp.dot(d2, we1rb[...], preferred_element_type=_f32) + be1[...])
    x = jnp.maximum(z1, 0.0)
    ef = jnp.maximum(jnp.dot(x, we2[...], preferred_element_type=_f32)
                     + be2[...], 0.0)
    h1 = jnp.maximum(jnp.dot(ef, wc1[...], preferred_element_type=_f32)
                     + bc1[...], 0.0)
    ce = jnp.dot(h1, wc2[...], preferred_element_type=_f32)
    out_ref[:, 0:H_DIM] = ef
    out_ref[:, H_DIM:H_DIM + CP] = (jnp.clip(diff * ce, -100.0, 100.0)
                                    + cntrow[...])


def _edge_mlp(g, we1rb, be1, we2, be2, wc1, bc1, wc2, cntrow):
    e_total = g.shape[0]
    grid = (e_total // EB,)
    full = lambda shape: pl.BlockSpec(shape, lambda i: (0, 0))
    return pl.pallas_call(
        _edge_body,
        grid=grid,
        in_specs=[
            pl.BlockSpec((EB, PW), lambda i: (i, 0)),
            full((CP, H_DIM)), full((1, H_DIM)),
            full((H_DIM, H_DIM)), full((1, H_DIM)),
            full((H_DIM, H_DIM)), full((1, H_DIM)), full((H_DIM, 1)),
            full((1, CP)),
        ],
        out_specs=pl.BlockSpec((EB, PW), lambda i: (i, 0)),
        out_shape=jax.ShapeDtypeStruct((e_total, PW), _f32),
    )(g, we1rb, be1, we2, be2, wc1, bc1, wc2, cntrow)


def _make_scatter_body(epw_a, epw_b):
    def body(pa_hbm, rowa_hbm, pb_hbm, rowb_hbm, out_hbm,
             idxv0, pv0, idxv1, pv1, shared,
             seml0, seml1, sema0, sema1):
        cid = lax.axis_index("c")
        sid = lax.axis_index("s")
        wid = sid * NC + cid
        slots = ((idxv0, pv0, seml0, sema0), (idxv1, pv1, seml1, sema1))

        def zrow(i, carry):
            for j in range(PW // 16):
                pv0[i, pl.ds(j * 16, 16)] = jnp.zeros((16,), _f32)
            return carry

        lax.fori_loop(0, CS, zrow, 0)

        def zcopy(i, carry):
            pltpu.sync_copy(pv0.at[pl.ds(0, CS)],
                            shared.at[pl.ds(sid * RPT + i * CS, CS)])
            return carry

        lax.fori_loop(0, RPT // CS, zcopy, 0)
        plsc.subcore_barrier()

        def run_phase(packed_hbm, row_hbm, epw):
            nch = epw // CS

            def issue(k, slot, wait_add):
                idxv, pv, seml, sema = slot
                base = wid * epw + k * CS
                if wait_add:
                    pltpu.make_async_copy(pv, shared.at[idxv], sema).wait()
                pltpu.sync_copy(row_hbm.at[pl.ds(base, CS)], idxv)
                pltpu.async_copy(packed_hbm.at[pl.ds(base, CS)], pv, seml)

            def proc(k, slot):
                idxv, pv, seml, sema = slot
                base = wid * epw + k * CS
                pltpu.make_async_copy(packed_hbm.at[pl.ds(base, CS)],
                                      pv, seml).wait()
                pltpu.async_copy(pv, shared.at[idxv], sema, add=True)

            issue(0, slots[0], False)
            issue(1, slots[1], False)
            npair = (nch - 2) // 2

            def pair(g, carry):
                proc(2 * g, slots[0])
                issue(2 * g + 2, slots[0], True)
                proc(2 * g + 1, slots[1])
                issue(2 * g + 3, slots[1], True)
                return carry

            lax.fori_loop(0, npair, pair, 0)
            for k in range(2 * npair, nch):
                proc(k, slots[k % 2])
                if k + 2 < nch:
                    issue(k + 2, slots[k % 2], True)
            pltpu.make_async_copy(pv0, shared.at[idxv0], sema0).wait()
            pltpu.make_async_copy(pv1, shared.at[idxv1], sema1).wait()

        run_phase(pa_hbm, rowa_hbm, epw_a)
        run_phase(pb_hbm, rowb_hbm, epw_b)
        plsc.subcore_barrier()
        pltpu.sync_copy(shared.at[pl.ds(sid * RPT, RPT)],
                        out_hbm.at[cid, pl.ds(sid * RPT, RPT)])

    return body


def _scatter(packed_a, row_a, packed_b, row_b):
    epw_a = packed_a.shape[0] // NW
    epw_b = packed_b.shape[0] // NW
    sk = functools.partial(
        pl.kernel,
        out_type=jax.ShapeDtypeStruct((NC, NP_NODES, PW), _f32),
        mesh=plsc.VectorSubcoreMesh(core_axis_name="c", subcore_axis_name="s",
                                    num_cores=NC, num_subcores=NS),
        scratch_types=[
            pltpu.VMEM((CS,), jnp.int32),
            pltpu.VMEM((CS, PW), _f32),
            pltpu.VMEM((CS,), jnp.int32),
            pltpu.VMEM((CS, PW), _f32),
            pltpu.VMEM_SHARED((NP_NODES, PW), _f32),
            pltpu.SemaphoreType.DMA,
            pltpu.SemaphoreType.DMA,
            pltpu.SemaphoreType.DMA,
            pltpu.SemaphoreType.DMA,
        ],
    )(_make_scatter_body(epw_a, epw_b))
    return sk(packed_a, row_a, packed_b, row_b)


def _node_body(agg0_ref, agg1_ref, hn_ref, wn1b, bn1, wn2, bn2,
               node_ref, force_ref):
    agg = agg0_ref[...] + agg1_ref[...]
    agg_e = agg[:, 0:H_DIM]
    t = jnp.maximum(hn_ref[...]
                    + jnp.dot(agg_e, wn1b[...], preferred_element_type=_f32)
                    + bn1[...], 0.0)
    node_ref[...] = jnp.dot(t, wn2[...], preferred_element_type=_f32) + bn2[...]
    tr = agg[:, H_DIM:H_DIM + 3]
    cnt = jnp.maximum(agg[:, H_DIM + CP - 1:H_DIM + CP], 1.0)
    force_ref[...] = tr / cnt


def _node_mlp(agg0, agg1, hn, wn1b, bn1, wn2, bn2):
    grid = (N_NODES // NB,)
    full = lambda shape: pl.BlockSpec(shape, lambda i: (0, 0))
    return pl.pallas_call(
        _node_body,
        grid=grid,
        in_specs=[
            pl.BlockSpec((NB, PW), lambda i: (i, 0)),
            pl.BlockSpec((NB, PW), lambda i: (i, 0)),
            pl.BlockSpec((NB, H_DIM), lambda i: (i, 0)),
            full((H_DIM, H_DIM)), full((1, H_DIM)),
            full((H_DIM, D_DIM)), full((1, D_DIM)),
        ],
        out_specs=[
            pl.BlockSpec((NB, D_DIM), lambda i: (i, 0)),
            pl.BlockSpec((NB, 3), lambda i: (i, 0)),
        ],
        out_shape=[
            jax.ShapeDtypeStruct((N_NODES, D_DIM), _f32),
            jax.ShapeDtypeStruct((N_NODES, 3), _f32),
        ],
    )(agg0, agg1, hn, wn1b, bn1, wn2, bn2)


def kernel(h, edge_index, coord, We1, be1, We2, be2, Wn1, bn1, Wn2, bn2,
           Wc1, bc1, Wc2, Wv1, bv1, Wv2, bv2):
    row = edge_index[0].astype(jnp.int32)
    col = edge_index[1].astype(jnp.int32)

    we1a = We1[:D_DIM]
    we1b = We1[D_DIM:2 * D_DIM]
    we1r = We1[2 * D_DIM:2 * D_DIM + 1]

    coord_p = jnp.zeros((N_NODES, CP), _f32).at[:, :3].set(coord)

    tr_tab, tb_tab, hn_tab, vel = _precompute(
        h, coord_p, we1a, we1b, Wn1[:D_DIM], Wv1,
        bv1.reshape(1, H_DIM), Wv2, bv2.reshape(1, 1))

    rw = row.reshape(NW, EPW)
    cw = col.reshape(NW, EPW)
    row_a = rw[:, :EPW_A].reshape(-1)
    col_a = cw[:, :EPW_A].reshape(-1)
    row_b = rw[:, EPW_A:].reshape(-1)
    col_b = cw[:, EPW_A:].reshape(-1)

    we1rb = jnp.broadcast_to(we1r, (CP, H_DIM))
    cntrow = jnp.zeros((1, CP), _f32).at[0, CP - 1].set(1.0)
    emlp = lambda g: _edge_mlp(g, we1rb,
                               be1.reshape(1, H_DIM), We2,
                               be2.reshape(1, H_DIM),
                               Wc1, bc1.reshape(1, H_DIM), Wc2, cntrow)

    g_a = _gather(tr_tab, tb_tab, row_a, col_a)
    packed_a = emlp(g_a)
    g_b = _gather(tr_tab, tb_tab, row_b, col_b)
    packed_b = emlp(g_b)

    agg = _scatter(packed_a, row_a, packed_b, row_b)

    node, force = _node_mlp(agg[0], agg[1], hn_tab,
                            Wn1[D_DIM:], bn1.reshape(1, H_DIM),
                            Wn2, bn2.reshape(1, D_DIM))
    return (vel, force, node)


# speedup vs baseline: 10.2711x; 1.1262x over previous
import functools

import jax
import jax.numpy as jnp
from jax import lax
from jax.experimental import pallas as pl
from jax.experimental.pallas import tpu as pltpu
from jax.experimental.pallas import tpu_sc as plsc

N_NODES = 10000
E_EDGES = 320000
D_DIM = 128
H_DIM = 64

NC, NS = 2, 16
NW = NC * NS
EPW = E_EDGES // NW
CG = 200
CS = 80
CP = 16
PW = 128
NP_NODES = 10240
RPT = NP_NODES // NS
NB = 2000
EB = 2000
EPW_A = 6000

_f32 = jnp.float32


def _pre_body(h_ref, c_ref, we1, wn1a, wv1, bv1, wv2, bv2,
              tr_ref, tb_ref, hn_ref, vel_ref):
    hb = h_ref[...]
    c3 = c_ref[...]
    pad = jnp.zeros((NB, PW - H_DIM - 3), _f32)
    a = jnp.dot(hb, we1[0:D_DIM, :], preferred_element_type=_f32)
    b = jnp.dot(hb, we1[D_DIM:2 * D_DIM, :], preferred_element_type=_f32)
    tr_ref[...] = jnp.concatenate([a, c3, pad], axis=1)
    tb_ref[...] = jnp.concatenate([b, -c3, pad], axis=1)
    hn_ref[...] = jnp.dot(hb, wn1a[...], preferred_element_type=_f32)
    t = jnp.maximum(jnp.dot(hb, wv1[...], preferred_element_type=_f32)
                    + bv1[...], 0.0)
    vel_ref[...] = jnp.dot(t, wv2[...], preferred_element_type=_f32) + bv2[...]


def _precompute(h, coord, we1, wn1a, wv1, bv1, wv2, bv2):
    n = h.shape[0]
    grid = (n // NB,)
    full = lambda shape: pl.BlockSpec(shape, lambda i: (0, 0))
    return pl.pallas_call(
        _pre_body,
        grid=grid,
        in_specs=[
            pl.BlockSpec((NB, D_DIM), lambda i: (i, 0)),
            pl.BlockSpec((NB, 3), lambda i: (i, 0)),
            full((2 * D_DIM + 1, H_DIM)), full((D_DIM, H_DIM)),
            full((D_DIM, H_DIM)), full((1, H_DIM)), full((H_DIM, 1)),
            full((1, 1)),
        ],
        out_specs=[
            pl.BlockSpec((NB, PW), lambda i: (i, 0)),
            pl.BlockSpec((NB, PW), lambda i: (i, 0)),
            pl.BlockSpec((NB, H_DIM), lambda i: (i, 0)),
            pl.BlockSpec((NB, 1), lambda i: (i, 0)),
        ],
        out_shape=[
            jax.ShapeDtypeStruct((n, PW), _f32),
            jax.ShapeDtypeStruct((n, PW), _f32),
            jax.ShapeDtypeStruct((n, H_DIM), _f32),
            jax.ShapeDtypeStruct((n, 1), _f32),
        ],
    )(h, coord, we1, wn1a, wv1, bv1, wv2, bv2)


def _make_gather_body(epw, off):
    nch = epw // CG
    npair = (nch - 2) // 2

    def body(tr_hbm, tb_hbm, row_hbm, col_hbm, g_hbm,
             rowv0, colv0, grv0, gcv0, rowv1, colv1, grv1, gcv1,
             semg0, semg1, semw0, semw1):
        wid = lax.axis_index("s") * NC + lax.axis_index("c")
        slots = ((rowv0, colv0, grv0, gcv0, semg0, semw0),
                 (rowv1, colv1, grv1, gcv1, semg1, semw1))

        def issue(k, slot, wait_wb):
            rowv, colv, grv, gcv, semg, semw = slot
            ibase = wid * EPW + off + k * CG
            base = wid * epw + k * CG
            pltpu.sync_copy(row_hbm.at[pl.ds(ibase, CG)], rowv)
            pltpu.sync_copy(col_hbm.at[pl.ds(ibase, CG)], colv)
            if wait_wb:
                pltpu.make_async_copy(gcv, g_hbm.at[pl.ds(base, CG)],
                                      semw).wait()
            pltpu.async_copy(tr_hbm.at[rowv], grv, semg)
            pltpu.async_copy(tb_hbm.at[colv], gcv, semg)

        def finish(k, slot):
            rowv, colv, grv, gcv, semg, semw = slot
            base = wid * epw + k * CG
            pltpu.make_async_copy(tr_hbm.at[rowv], grv, semg).wait()
            pltpu.make_async_copy(tb_hbm.at[colv], gcv, semg).wait()

            def add_row(r, carry):
                for j in range(PW // 16):
                    sl = pl.ds(j * 16, 16)
                    gcv[r, sl] = grv[r, sl] + gcv[r, sl]
                return carry

            lax.fori_loop(0, CG, add_row, 0)
            pltpu.async_copy(gcv, g_hbm.at[pl.ds(base, CG)], semw)

        issue(0, slots[0], False)
        issue(1, slots[1], False)

        def pair(g, carry):
            finish(2 * g, slots[0])
            issue(2 * g + 2, slots[0], True)
            finish(2 * g + 1, slots[1])
            issue(2 * g + 3, slots[1], True)
            return carry

        lax.fori_loop(0, npair, pair, 0)
        for k in range(2 * npair, nch):
            finish(k, slots[k % 2])
            if k + 2 < nch:
                issue(k + 2, slots[k % 2], True)
        pltpu.make_async_copy(
            gcv0, g_hbm.at[pl.ds(0, CG)], semw0).wait()
        pltpu.make_async_copy(
            gcv1, g_hbm.at[pl.ds(0, CG)], semw1).wait()

    return body


def _gather(tr_tab, tb_tab, row, col, epw, off):
    gk = functools.partial(
        pl.kernel,
        out_type=jax.ShapeDtypeStruct((epw * NW, PW), _f32),
        mesh=plsc.VectorSubcoreMesh(core_axis_name="c", subcore_axis_name="s",
                                    num_cores=NC, num_subcores=NS),
        scratch_types=[
            pltpu.VMEM((CG,), jnp.int32),
            pltpu.VMEM((CG,), jnp.int32),
            pltpu.VMEM((CG, PW), _f32),
            pltpu.VMEM((CG, PW), _f32),
            pltpu.VMEM((CG,), jnp.int32),
            pltpu.VMEM((CG,), jnp.int32),
            pltpu.VMEM((CG, PW), _f32),
            pltpu.VMEM((CG, PW), _f32),
            pltpu.SemaphoreType.DMA,
            pltpu.SemaphoreType.DMA,
            pltpu.SemaphoreType.DMA,
            pltpu.SemaphoreType.DMA,
        ],
    )(_make_gather_body(epw, off))
    return gk(tr_tab, tb_tab, row, col)


def _edge_body(g_ref, we1, be1, we2, be2, wc1, bc1, wc2, out_ref):
    g = g_ref[...]
    diff = g[:, H_DIM:H_DIM + CP]
    d2 = diff * diff
    we1rb = jnp.broadcast_to(we1[2 * D_DIM:2 * D_DIM + 1, :], (CP, H_DIM))
    z1 = (g[:, 0:H_DIM]
          + jnp.dot(d2, we1rb, preferred_element_type=_f32) + be1[...])
    x = jnp.maximum(z1, 0.0)
    ef = jnp.maximum(jnp.dot(x, we2[...], preferred_element_type=_f32)
                     + be2[...], 0.0)
    h1 = jnp.maximum(jnp.dot(ef, wc1[...], preferred_element_type=_f32)
                     + bc1[...], 0.0)
    ce = jnp.dot(h1, wc2[...], preferred_element_type=_f32)
    cnt = (lax.broadcasted_iota(jnp.int32, (EB, CP), 1) == CP - 1
           ).astype(_f32)
    out_ref[:, 0:H_DIM] = ef
    out_ref[:, H_DIM:H_DIM + CP] = (jnp.clip(diff * ce, -100.0, 100.0)
                                    + cnt)


def _edge_mlp(g, we1, be1, we2, be2, wc1, bc1, wc2):
    e_total = g.shape[0]
    grid = (e_total // EB,)
    full = lambda shape: pl.BlockSpec(shape, lambda i: (0, 0))
    return pl.pallas_call(
        _edge_body,
        grid=grid,
        in_specs=[
            pl.BlockSpec((EB, PW), lambda i: (i, 0)),
            full((2 * D_DIM + 1, H_DIM)), full((1, H_DIM)),
            full((H_DIM, H_DIM)), full((1, H_DIM)),
            full((H_DIM, H_DIM)), full((1, H_DIM)), full((H_DIM, 1)),
        ],
        out_specs=pl.BlockSpec((EB, PW), lambda i: (i, 0)),
        out_shape=jax.ShapeDtypeStruct((e_total, PW), _f32),
    )(g, we1, be1, we2, be2, wc1, bc1, wc2)


def _make_scatter_body(epw, off):
    def body(packed_hbm, row_hbm, out_hbm,
             idxv0, pv0, idxv1, pv1, shared,
             seml0, seml1, sema0, sema1):
        cid = lax.axis_index("c")
        sid = lax.axis_index("s")
        wid = sid * NC + cid
        slots = ((idxv0, pv0, seml0, sema0), (idxv1, pv1, seml1, sema1))

        def zrow(i, carry):
            for j in range(PW // 16):
                pv0[i, pl.ds(j * 16, 16)] = jnp.zeros((16,), _f32)
            return carry

        lax.fori_loop(0, CS, zrow, 0)

        def zcopy(i, carry):
            pltpu.sync_copy(pv0.at[pl.ds(0, CS)],
                            shared.at[pl.ds(sid * RPT + i * CS, CS)])
            return carry

        lax.fori_loop(0, RPT // CS, zcopy, 0)
        plsc.subcore_barrier()

        nch = epw // CS

        def issue(k, slot, wait_add):
            idxv, pv, seml, sema = slot
            ibase = wid * EPW + off + k * CS
            base = wid * epw + k * CS
            if wait_add:
                pltpu.make_async_copy(pv, shared.at[idxv], sema).wait()
            pltpu.sync_copy(row_hbm.at[pl.ds(ibase, CS)], idxv)
            pltpu.async_copy(packed_hbm.at[pl.ds(base, CS)], pv, seml)

        def proc(k, slot):
            idxv, pv, seml, sema = slot
            base = wid * epw + k * CS
            pltpu.make_async_copy(packed_hbm.at[pl.ds(base, CS)],
                                  pv, seml).wait()
            pltpu.async_copy(pv, shared.at[idxv], sema, add=True)

        issue(0, slots[0], False)
        issue(1, slots[1], False)
        npair = (nch - 2) // 2

        def pair(g, carry):
            proc(2 * g, slots[0])
            issue(2 * g + 2, slots[0], True)
            proc(2 * g + 1, slots[1])
            issue(2 * g + 3, slots[1], True)
            return carry

        lax.fori_loop(0, npair, pair, 0)
        for k in range(2 * npair, nch):
            proc(k, slots[k % 2])
            if k + 2 < nch:
                issue(k + 2, slots[k % 2], True)
        pltpu.make_async_copy(pv0, shared.at[idxv0], sema0).wait()
        pltpu.make_async_copy(pv1, shared.at[idxv1], sema1).wait()
        plsc.subcore_barrier()
        pltpu.sync_copy(shared.at[pl.ds(sid * RPT, RPT)],
                        out_hbm.at[cid, pl.ds(sid * RPT, RPT)])

    return body


def _scatter(packed, row, epw, off):
    sk = functools.partial(
        pl.kernel,
        out_type=jax.ShapeDtypeStruct((NC, NP_NODES, PW), _f32),
        mesh=plsc.VectorSubcoreMesh(core_axis_name="c", subcore_axis_name="s",
                                    num_cores=NC, num_subcores=NS),
        scratch_types=[
            pltpu.VMEM((CS,), jnp.int32),
            pltpu.VMEM((CS, PW), _f32),
            pltpu.VMEM((CS,), jnp.int32),
            pltpu.VMEM((CS, PW), _f32),
            pltpu.VMEM_SHARED((NP_NODES, PW), _f32),
            pltpu.SemaphoreType.DMA,
            pltpu.SemaphoreType.DMA,
            pltpu.SemaphoreType.DMA,
            pltpu.SemaphoreType.DMA,
        ],
    )(_make_scatter_body(epw, off))
    return sk(packed, row)


def _node_body(agg0_ref, agg1_ref, agg2_ref, agg3_ref, hn_ref,
               wn1b, bn1, wn2, bn2, node_ref, force_ref):
    agg = ((agg0_ref[...] + agg1_ref[...])
           + (agg2_ref[...] + agg3_ref[...]))
    agg_e = agg[:, 0:H_DIM]
    t = jnp.maximum(hn_ref[...]
                    + jnp.dot(agg_e, wn1b[...], preferred_element_type=_f32)
                    + bn1[...], 0.0)
    node_ref[...] = jnp.dot(t, wn2[...], preferred_element_type=_f32) + bn2[...]
    tr = agg[:, H_DIM:H_DIM + 3]
    cnt = jnp.maximum(agg[:, H_DIM + CP - 1:H_DIM + CP], 1.0)
    force_ref[...] = tr / cnt


def _node_mlp(agg0, agg1, agg2, agg3, hn, wn1b, bn1, wn2, bn2):
    grid = (N_NODES // NB,)
    full = lambda shape: pl.BlockSpec(shape, lambda i: (0, 0))
    return pl.pallas_call(
        _node_body,
        grid=grid,
        in_specs=[
            pl.BlockSpec((NB, PW), lambda i: (i, 0)),
            pl.BlockSpec((NB, PW), lambda i: (i, 0)),
            pl.BlockSpec((NB, PW), lambda i: (i, 0)),
            pl.BlockSpec((NB, PW), lambda i: (i, 0)),
            pl.BlockSpec((NB, H_DIM), lambda i: (i, 0)),
            full((H_DIM, H_DIM)), full((1, H_DIM)),
            full((H_DIM, D_DIM)), full((1, D_DIM)),
        ],
        out_specs=[
            pl.BlockSpec((NB, D_DIM), lambda i: (i, 0)),
            pl.BlockSpec((NB, 3), lambda i: (i, 0)),
        ],
        out_shape=[
            jax.ShapeDtypeStruct((N_NODES, D_DIM), _f32),
            jax.ShapeDtypeStruct((N_NODES, 3), _f32),
        ],
    )(agg0, agg1, agg2, agg3, hn, wn1b, bn1, wn2, bn2)


def kernel(h, edge_index, coord, We1, be1, We2, be2, Wn1, bn1, Wn2, bn2,
           Wc1, bc1, Wc2, Wv1, bv1, Wv2, bv2):
    row = edge_index[0]
    col = edge_index[1]
    if row.dtype != jnp.int32:
        row = row.astype(jnp.int32)
        col = col.astype(jnp.int32)

    tr_tab, tb_tab, hn_tab, vel = _precompute(
        h, coord, We1, Wn1[:D_DIM], Wv1,
        bv1.reshape(1, H_DIM), Wv2, bv2.reshape(1, 1))

    emlp = lambda g: _edge_mlp(g, We1,
                               be1.reshape(1, H_DIM), We2,
                               be2.reshape(1, H_DIM),
                               Wc1, bc1.reshape(1, H_DIM), Wc2)

    g_a = _gather(tr_tab, tb_tab, row, col, EPW_A, 0)
    packed_a = emlp(g_a)
    g_b = _gather(tr_tab, tb_tab, row, col, EPW - EPW_A, EPW_A)
    packed_b = emlp(g_b)

    agg_a = _scatter(packed_a, row, EPW_A, 0)
    agg_b = _scatter(packed_b, row, EPW - EPW_A, EPW_A)

    node, force = _node_mlp(agg_a[0], agg_a[1], agg_b[0], agg_b[1], hn_tab,
                            Wn1[D_DIM:], bn1.reshape(1, H_DIM),
                            Wn2, bn2.reshape(1, D_DIM))
    return (vel, force, node)


# speedup vs baseline: 10.4471x; 1.0171x over previous
import functools

import jax
import jax.numpy as jnp
from jax import lax
from jax.experimental import pallas as pl
from jax.experimental.pallas import tpu as pltpu
from jax.experimental.pallas import tpu_sc as plsc

N_NODES = 10000
E_EDGES = 320000
D_DIM = 128
H_DIM = 64

NC, NS = 2, 16
NW = NC * NS
EPW = E_EDGES // NW
CG = 200
CS = 80
CP = 16
PW = 128
NP_NODES = 10240
RPT = NP_NODES // NS
NB = 2000
EB = 2000
EPW_A = 6000

_f32 = jnp.float32


def _pre_body(h_ref, c_ref, we1, wn1, wv1, bv1, wv2, bv2,
              tr_ref, tb_ref, hn_ref, vel_ref):
    hb = h_ref[...]
    c3 = c_ref[...]
    pad = jnp.zeros((NB, PW - H_DIM - 3), _f32)
    a = jnp.dot(hb, we1[0:D_DIM, :], preferred_element_type=_f32)
    b = jnp.dot(hb, we1[D_DIM:2 * D_DIM, :], preferred_element_type=_f32)
    tr_ref[...] = jnp.concatenate([a, c3, pad], axis=1)
    tb_ref[...] = jnp.concatenate([b, -c3, pad], axis=1)
    hn_ref[...] = jnp.dot(hb, wn1[0:D_DIM, :], preferred_element_type=_f32)
    t = jnp.maximum(jnp.dot(hb, wv1[...], preferred_element_type=_f32)
                    + bv1[...], 0.0)
    vel_ref[...] = jnp.dot(t, wv2[...], preferred_element_type=_f32) + bv2[...]


def _precompute(h, coord, we1, wn1, wv1, bv1, wv2, bv2):
    n = h.shape[0]
    grid = (n // NB,)
    full = lambda shape: pl.BlockSpec(shape, lambda i: (0, 0))
    return pl.pallas_call(
        _pre_body,
        grid=grid,
        in_specs=[
            pl.BlockSpec((NB, D_DIM), lambda i: (i, 0)),
            pl.BlockSpec((NB, 3), lambda i: (i, 0)),
            full((2 * D_DIM + 1, H_DIM)), full((D_DIM + H_DIM, H_DIM)),
            full((D_DIM, H_DIM)), full((1, H_DIM)), full((H_DIM, 1)),
            full((1, 1)),
        ],
        out_specs=[
            pl.BlockSpec((NB, PW), lambda i: (i, 0)),
            pl.BlockSpec((NB, PW), lambda i: (i, 0)),
            pl.BlockSpec((NB, H_DIM), lambda i: (i, 0)),
            pl.BlockSpec((NB, 1), lambda i: (i, 0)),
        ],
        out_shape=[
            jax.ShapeDtypeStruct((n, PW), _f32),
            jax.ShapeDtypeStruct((n, PW), _f32),
            jax.ShapeDtypeStruct((n, H_DIM), _f32),
            jax.ShapeDtypeStruct((n, 1), _f32),
        ],
    )(h, coord, we1, wn1, wv1, bv1, wv2, bv2)


def _make_gather_body(epw, off):
    nch = epw // CG
    npair = (nch - 2) // 2

    def body(tr_hbm, tb_hbm, row_hbm, col_hbm, g_hbm,
             rowv0, colv0, grv0, gcv0, rowv1, colv1, grv1, gcv1,
             semg0, semg1, semw0, semw1):
        wid = lax.axis_index("s") * NC + lax.axis_index("c")
        slots = ((rowv0, colv0, grv0, gcv0, semg0, semw0),
                 (rowv1, colv1, grv1, gcv1, semg1, semw1))

        def issue(k, slot, wait_wb):
            rowv, colv, grv, gcv, semg, semw = slot
            ibase = wid * EPW + off + k * CG
            base = wid * epw + k * CG
            pltpu.sync_copy(row_hbm.at[pl.ds(ibase, CG)], rowv)
            pltpu.sync_copy(col_hbm.at[pl.ds(ibase, CG)], colv)
            if wait_wb:
                pltpu.make_async_copy(gcv, g_hbm.at[pl.ds(base, CG)],
                                      semw).wait()
            pltpu.async_copy(tr_hbm.at[rowv], grv, semg)
            pltpu.async_copy(tb_hbm.at[colv], gcv, semg)

        def finish(k, slot):
            rowv, colv, grv, gcv, semg, semw = slot
            base = wid * epw + k * CG
            pltpu.make_async_copy(tr_hbm.at[rowv], grv, semg).wait()
            pltpu.make_async_copy(tb_hbm.at[colv], gcv, semg).wait()

            def add_row(r, carry):
                for j in range(PW // 16):
                    sl = pl.ds(j * 16, 16)
                    gcv[r, sl] = grv[r, sl] + gcv[r, sl]
                return carry

            lax.fori_loop(0, CG, add_row, 0)
            pltpu.async_copy(gcv, g_hbm.at[pl.ds(base, CG)], semw)

        issue(0, slots[0], False)
        issue(1, slots[1], False)

        def pair(g, carry):
            finish(2 * g, slots[0])
            issue(2 * g + 2, slots[0], True)
            finish(2 * g + 1, slots[1])
            issue(2 * g + 3, slots[1], True)
            return carry

        lax.fori_loop(0, npair, pair, 0)
        for k in range(2 * npair, nch):
            finish(k, slots[k % 2])
            if k + 2 < nch:
                issue(k + 2, slots[k % 2], True)
        pltpu.make_async_copy(
            gcv0, g_hbm.at[pl.ds(0, CG)], semw0).wait()
        pltpu.make_async_copy(
            gcv1, g_hbm.at[pl.ds(0, CG)], semw1).wait()

    return body


def _gather(tr_tab, tb_tab, row, col, epw, off):
    gk = functools.partial(
        pl.kernel,
        out_type=jax.ShapeDtypeStruct((epw * NW, PW), _f32),
        mesh=plsc.VectorSubcoreMesh(core_axis_name="c", subcore_axis_name="s",
                                    num_cores=NC, num_subcores=NS),
        scratch_types=[
            pltpu.VMEM((CG,), jnp.int32),
            pltpu.VMEM((CG,), jnp.int32),
            pltpu.VMEM((CG, PW), _f32),
            pltpu.VMEM((CG, PW), _f32),
            pltpu.VMEM((CG,), jnp.int32),
            pltpu.VMEM((CG,), jnp.int32),
            pltpu.VMEM((CG, PW), _f32),
            pltpu.VMEM((CG, PW), _f32),
            pltpu.SemaphoreType.DMA,
            pltpu.SemaphoreType.DMA,
            pltpu.SemaphoreType.DMA,
            pltpu.SemaphoreType.DMA,
        ],
    )(_make_gather_body(epw, off))
    return gk(tr_tab, tb_tab, row, col)


def _edge_body(g_ref, we1, be1, we2, be2, wc1, bc1, wc2, out_ref):
    g = g_ref[...]
    diff = g[:, H_DIM:H_DIM + CP]
    d2 = diff * diff
    we1rb = jnp.broadcast_to(we1[2 * D_DIM:2 * D_DIM + 1, :], (CP, H_DIM))
    z1 = (g[:, 0:H_DIM]
          + jnp.dot(d2, we1rb, preferred_element_type=_f32) + be1[...])
    x = jnp.maximum(z1, 0.0)
    ef = jnp.maximum(jnp.dot(x, we2[...], preferred_element_type=_f32)
                     + be2[...], 0.0)
    h1 = jnp.maximum(jnp.dot(ef, wc1[...], preferred_element_type=_f32)
                     + bc1[...], 0.0)
    ce = jnp.dot(h1, wc2[...], preferred_element_type=_f32)
    cnt = (lax.broadcasted_iota(jnp.int32, (EB, CP), 1) == CP - 1
           ).astype(_f32)
    out_ref[:, 0:H_DIM] = ef
    out_ref[:, H_DIM:H_DIM + CP] = (jnp.clip(diff * ce, -100.0, 100.0)
                                    + cnt)


def _edge_mlp(g, we1, be1, we2, be2, wc1, bc1, wc2):
    e_total = g.shape[0]
    grid = (e_total // EB,)
    full = lambda shape: pl.BlockSpec(shape, lambda i: (0, 0))
    return pl.pallas_call(
        _edge_body,
        grid=grid,
        in_specs=[
            pl.BlockSpec((EB, PW), lambda i: (i, 0)),
            full((2 * D_DIM + 1, H_DIM)), full((1, H_DIM)),
            full((H_DIM, H_DIM)), full((1, H_DIM)),
            full((H_DIM, H_DIM)), full((1, H_DIM)), full((H_DIM, 1)),
        ],
        out_specs=pl.BlockSpec((EB, PW), lambda i: (i, 0)),
        out_shape=jax.ShapeDtypeStruct((e_total, PW), _f32),
    )(g, we1, be1, we2, be2, wc1, bc1, wc2)


def _make_scatter_body(epw, off):
    def body(packed_hbm, row_hbm, out_hbm,
             idxv0, pv0, idxv1, pv1, shared,
             seml0, seml1, sema0, sema1):
        cid = lax.axis_index("c")
        sid = lax.axis_index("s")
        wid = sid * NC + cid
        slots = ((idxv0, pv0, seml0, sema0), (idxv1, pv1, seml1, sema1))

        def zrow(i, carry):
            for j in range(PW // 16):
                pv0[i, pl.ds(j * 16, 16)] = jnp.zeros((16,), _f32)
            return carry

        lax.fori_loop(0, CS, zrow, 0)

        def zcopy(i, carry):
            pltpu.sync_copy(pv0.at[pl.ds(0, CS)],
                            shared.at[pl.ds(sid * RPT + i * CS, CS)])
            return carry

        lax.fori_loop(0, RPT // CS, zcopy, 0)
        plsc.subcore_barrier()

        nch = epw // CS

        def issue(k, slot, wait_add):
            idxv, pv, seml, sema = slot
            ibase = wid * EPW + off + k * CS
            base = wid * epw + k * CS
            if wait_add:
                pltpu.make_async_copy(pv, shared.at[idxv], sema).wait()
            pltpu.sync_copy(row_hbm.at[pl.ds(ibase, CS)], idxv)
            pltpu.async_copy(packed_hbm.at[pl.ds(base, CS)], pv, seml)

        def proc(k, slot):
            idxv, pv, seml, sema = slot
            base = wid * epw + k * CS
            pltpu.make_async_copy(packed_hbm.at[pl.ds(base, CS)],
                                  pv, seml).wait()
            pltpu.async_copy(pv, shared.at[idxv], sema, add=True)

        issue(0, slots[0], False)
        issue(1, slots[1], False)
        npair = (nch - 2) // 2

        def pair(g, carry):
            proc(2 * g, slots[0])
            issue(2 * g + 2, slots[0], True)
            proc(2 * g + 1, slots[1])
            issue(2 * g + 3, slots[1], True)
            return carry

        lax.fori_loop(0, npair, pair, 0)
        for k in range(2 * npair, nch):
            proc(k, slots[k % 2])
            if k + 2 < nch:
                issue(k + 2, slots[k % 2], True)
        pltpu.make_async_copy(pv0, shared.at[idxv0], sema0).wait()
        pltpu.make_async_copy(pv1, shared.at[idxv1], sema1).wait()
        plsc.subcore_barrier()
        pltpu.sync_copy(shared.at[pl.ds(sid * RPT, RPT)],
                        out_hbm.at[cid, pl.ds(sid * RPT, RPT)])

    return body


def _scatter(packed, row, epw, off):
    sk = functools.partial(
        pl.kernel,
        out_type=jax.ShapeDtypeStruct((NC, NP_NODES, PW), _f32),
        mesh=plsc.VectorSubcoreMesh(core_axis_name="c", subcore_axis_name="s",
                                    num_cores=NC, num_subcores=NS),
        scratch_types=[
            pltpu.VMEM((CS,), jnp.int32),
            pltpu.VMEM((CS, PW), _f32),
            pltpu.VMEM((CS,), jnp.int32),
            pltpu.VMEM((CS, PW), _f32),
            pltpu.VMEM_SHARED((NP_NODES, PW), _f32),
            pltpu.SemaphoreType.DMA,
            pltpu.SemaphoreType.DMA,
            pltpu.SemaphoreType.DMA,
            pltpu.SemaphoreType.DMA,
        ],
    )(_make_scatter_body(epw, off))
    return sk(packed, row)


def _node_body(agg0_ref, agg1_ref, agg2_ref, agg3_ref, hn_ref,
               wn1, bn1, wn2, bn2, node_ref, force_ref):
    agg = ((agg0_ref[0] + agg1_ref[0])
           + (agg2_ref[0] + agg3_ref[0]))
    agg_e = agg[:, 0:H_DIM]
    t = jnp.maximum(hn_ref[...]
                    + jnp.dot(agg_e, wn1[D_DIM:D_DIM + H_DIM, :],
                              preferred_element_type=_f32)
                    + bn1[...], 0.0)
    node_ref[...] = jnp.dot(t, wn2[...], preferred_element_type=_f32) + bn2[...]
    tr = agg[:, H_DIM:H_DIM + 3]
    cnt = jnp.maximum(agg[:, H_DIM + CP - 1:H_DIM + CP], 1.0)
    force_ref[...] = tr / cnt


def _node_mlp(agg_a, agg_b, hn, wn1, bn1, wn2, bn2):
    grid = (N_NODES // NB,)
    full = lambda shape: pl.BlockSpec(shape, lambda i: (0, 0))
    return pl.pallas_call(
        _node_body,
        grid=grid,
        in_specs=[
            pl.BlockSpec((1, NB, PW), lambda i: (0, i, 0)),
            pl.BlockSpec((1, NB, PW), lambda i: (1, i, 0)),
            pl.BlockSpec((1, NB, PW), lambda i: (0, i, 0)),
            pl.BlockSpec((1, NB, PW), lambda i: (1, i, 0)),
            pl.BlockSpec((NB, H_DIM), lambda i: (i, 0)),
            full((D_DIM + H_DIM, H_DIM)), full((1, H_DIM)),
            full((H_DIM, D_DIM)), full((1, D_DIM)),
        ],
        out_specs=[
            pl.BlockSpec((NB, D_DIM), lambda i: (i, 0)),
            pl.BlockSpec((NB, 3), lambda i: (i, 0)),
        ],
        out_shape=[
            jax.ShapeDtypeStruct((N_NODES, D_DIM), _f32),
            jax.ShapeDtypeStruct((N_NODES, 3), _f32),
        ],
    )(agg_a, agg_a, agg_b, agg_b, hn, wn1, bn1, wn2, bn2)


def kernel(h, edge_index, coord, We1, be1, We2, be2, Wn1, bn1, Wn2, bn2,
           Wc1, bc1, Wc2, Wv1, bv1, Wv2, bv2):
    row = edge_index[0]
    col = edge_index[1]
    if row.dtype != jnp.int32:
        row = row.astype(jnp.int32)
        col = col.astype(jnp.int32)

    tr_tab, tb_tab, hn_tab, vel = _precompute(
        h, coord, We1, Wn1, Wv1,
        bv1.reshape(1, H_DIM), Wv2, bv2.reshape(1, 1))

    emlp = lambda g: _edge_mlp(g, We1,
                               be1.reshape(1, H_DIM), We2,
                               be2.reshape(1, H_DIM),
                               Wc1, bc1.reshape(1, H_DIM), Wc2)

    g_a = _gather(tr_tab, tb_tab, row, col, EPW_A, 0)
    packed_a = emlp(g_a)
    g_b = _gather(tr_tab, tb_tab, row, col, EPW - EPW_A, EPW_A)
    packed_b = emlp(g_b)

    agg_a = _scatter(packed_a, row, EPW_A, 0)
    agg_b = _scatter(packed_b, row, EPW - EPW_A, EPW_A)

    node, force = _node_mlp(agg_a, agg_b, hn_tab,
                            Wn1, bn1.reshape(1, H_DIM),
                            Wn2, bn2.reshape(1, D_DIM))
    return (vel, force, node)
